# baseline (speedup 1.0000x reference)
import jax
import jax.numpy as jnp
from jax import lax
from jax.experimental import pallas as pl
from jax.experimental.pallas import tpu as pltpu

_B = 16
_C = 16
_NPB = 5456
_N = _B * _NPB
_K = int(_N * 0.01)
_H0 = 64
_W0 = 64
_N1 = 1024
_NSM = 336
_OFF1 = 4096
_OFFSM = 5120


def _smooth_l1(x, y):
    d = jnp.abs(x - y)
    return jnp.where(d < 1.0, 0.5 * d * d, d - 0.5)


def _vbody(c0_ref, c1_ref, csm_ref, v0_ref, v1_ref, vsm_ref):
    v0_ref[...] = jax.nn.sigmoid(jnp.max(c0_ref[...], axis=1))
    v1_ref[...] = jax.nn.sigmoid(jnp.max(c1_ref[...], axis=1))[:, None, :]
    vsm_ref[...] = jax.nn.sigmoid(jnp.max(csm_ref[...], axis=1))[:, None, :]


def _softplus_p(x):
    e = jnp.exp(-jnp.abs(x))
    a = 1.0 + e
    p = jnp.where(x >= 0.0, 1.0, e) / a
    sp = jnp.maximum(x, 0.0) + jnp.log(a)
    return sp, p


def _lbody(*refs):
    (v0, v1, vsm,
     tc0, sc0, tb0, ta0, tt0, sb0, sa0, st0,
     tc1, sc1, tb1, tt1, sb1, st1,
     tcs, scs, tbs, tts, sbs, sts,
     out_cls, out_bbox, out_ctr, sdi, sdf) = refs

    i = pl.program_id(0)

    def keyed(vref):
        return lax.bitcast_convert_type(vref[...], jnp.int32)

    def idx0_full(shape):
        return (lax.broadcasted_iota(jnp.int32, shape, 0) * _NPB
                + lax.broadcasted_iota(jnp.int32, shape, 1) * _W0
                + lax.broadcasted_iota(jnp.int32, shape, 2))

    def idxp_full(shape, off):
        return (lax.broadcasted_iota(jnp.int32, shape, 0) * _NPB + off
                + lax.broadcasted_iota(jnp.int32, shape, 2))

    @pl.when(i == 0)
    def _search():
        k0 = keyed(v0)
        k1 = keyed(v1)
        ksm = keyed(vsm)

        def cnt_gt(x):
            return (jnp.sum((k0 > x).astype(jnp.int32))
                    + jnp.sum((k1 > x).astype(jnp.int32))
                    + jnp.sum((ksm > x).astype(jnp.int32)))

        def bstep(_, lohi):
            lo, hi = lohi
            mid = lo + (hi - lo) // 2
            take_hi = cnt_gt(mid) < _K
            return (jnp.where(take_hi, lo, mid), jnp.where(take_hi, mid, hi))

        _, t_key = lax.fori_loop(0, 31, bstep,
                                 (jnp.int32(-1), jnp.int32(0x3F800000)))
        r = _K - cnt_gt(t_key)

        i0 = idx0_full(k0.shape)
        i1 = idxp_full(k1.shape, _OFF1)
        ism = idxp_full(ksm.shape, _OFFSM)

        def cnt_eq_le(x):
            return (jnp.sum(((k0 == t_key) & (i0 <= x)).astype(jnp.int32))
                    + jnp.sum(((k1 == t_key) & (i1 <= x)).astype(jnp.int32))
                    + jnp.sum(((ksm == t_key) & (ism <= x)).astype(jnp.int32)))

        def istep(_, lohi):
            lo, hi = lohi
            mid = lo + (hi - lo) // 2
            enough = cnt_eq_le(mid) >= r
            return (jnp.where(enough, lo, mid), jnp.where(enough, mid, hi))

        _, i_star = lax.fori_loop(0, 17, istep,
                                  (jnp.int32(-1), jnp.int32(_N - 1)))

        t_val = lax.bitcast_convert_type(t_key, jnp.float32)
        fg = (jnp.sum(jnp.where(k0 > t_key, v0[...], 0.0))
              + jnp.sum(jnp.where(k1 > t_key, v1[...], 0.0))
              + jnp.sum(jnp.where(ksm > t_key, vsm[...], 0.0)))
        sdi[0] = t_key
        sdi[1] = i_star
        sdf[0] = fg + t_val * r.astype(jnp.float32)
        sdf[1] = jnp.float32(0.0)
        sdf[2] = jnp.float32(0.0)
        sdf[3] = jnp.float32(0.0)

    @pl.when(i > 0)
    def _losses():
        b = i - 1
        t_key = sdi[0]
        i_star = sdi[1]
        acc_cls = jnp.float32(0.0)
        acc_bbox = jnp.float32(0.0)
        acc_ctr = jnp.float32(0.0)

        def group(v_blk, idx, t_cls_v, s_cls_v, t_bb5, s_bb5, t_ct, s_ct):
            nonlocal acc_cls, acc_bbox, acc_ctr
            key = lax.bitcast_convert_type(v_blk, jnp.int32)
            mask = (key > t_key) | ((key == t_key) & (idx <= i_star))

            sp, p = _softplus_p(s_cls_v)
            t = jax.nn.sigmoid(t_cls_v)
            d = t - p
            pos = (sp - t * s_cls_v) * (d * d)
            neg = sp * (p * p)
            if s_cls_v.ndim == 4:
                mcls = mask[:, None, :, :]
            else:
                mcls = mask
            acc_cls = acc_cls + jnp.sum(jnp.where(mcls, pos, neg))

            sl1 = jnp.sum(_smooth_l1(s_bb5, t_bb5), axis=1)
            w = jax.nn.sigmoid(t_ct)
            macc = mask[:, 0] if s_cls_v.ndim == 3 else mask
            acc_bbox = acc_bbox + jnp.sum(jnp.where(macc, sl1 * w, 0.0))

            sps, _ = _softplus_p(s_ct)
            acc_ctr = acc_ctr + jnp.sum(jnp.where(macc, sps - w * s_ct, 0.0))

        v_blk0 = v0[pl.ds(b, 1), :, :]
        idx0 = b * _NPB + (
            lax.broadcasted_iota(jnp.int32, v_blk0.shape, 1) * _W0
            + lax.broadcasted_iota(jnp.int32, v_blk0.shape, 2))
        bb5_t0 = jnp.concatenate([tb0[...], ta0[...]], axis=1)
        bb5_s0 = jnp.concatenate([sb0[...], sa0[...]], axis=1)
        group(v_blk0, idx0, tc0[...], sc0[...], bb5_t0, bb5_s0,
              tt0[...][:, 0], st0[...][:, 0])

        v_blk1 = v1[pl.ds(b, 1), :, :]
        idx1 = (b * _NPB + _OFF1
                + lax.broadcasted_iota(jnp.int32, v_blk1.shape, 2))
        group(v_blk1, idx1, tc1[...], sc1[...], tb1[...], sb1[...],
              tt1[...][:, 0], st1[...][:, 0])

        v_blks = vsm[pl.ds(b, 1), :, :]
        idxs = (b * _NPB + _OFFSM
                + lax.broadcasted_iota(jnp.int32, v_blks.shape, 2))
        group(v_blks, idxs, tcs[...], scs[...], tbs[...], sbs[...],
              tts[...][:, 0], sts[...][:, 0])

        sdf[1] += acc_cls
        sdf[2] += acc_bbox
        sdf[3] += acc_ctr

    @pl.when(i == pl.num_programs(0) - 1)
    def _finish():
        out_cls[0, 0] = sdf[1] / sdf[0]
        out_bbox[0, 0] = sdf[2] / jnp.float32(_K * 5)
        out_ctr[0, 0] = sdf[3] / jnp.float32(_K)


def _pack_side(cls_l, bbox_l, angle_l, ctr_l):
    c1 = cls_l[1].reshape(_B, _C, _N1)
    csm = jnp.concatenate([x.reshape(_B, _C, -1) for x in cls_l[2:]], axis=2)
    bb1 = jnp.concatenate([bbox_l[1].reshape(_B, 4, _N1),
                           angle_l[1].reshape(_B, 1, _N1)], axis=1)
    bbsm = jnp.concatenate(
        [jnp.concatenate([x.reshape(_B, 4, -1), y.reshape(_B, 1, -1)], axis=1)
         for x, y in zip(bbox_l[2:], angle_l[2:])], axis=2)
    ct1 = ctr_l[1].reshape(_B, 1, _N1)
    ctsm = jnp.concatenate([x.reshape(_B, 1, -1) for x in ctr_l[2:]], axis=2)
    return c1, csm, bb1, bbsm, ct1, ctsm


@jax.jit
def kernel(
    t_cls_0, t_cls_1, t_cls_2, t_cls_3, t_cls_4,
    t_bbox_0, t_bbox_1, t_bbox_2, t_bbox_3, t_bbox_4,
    t_angle_0, t_angle_1, t_angle_2, t_angle_3, t_angle_4,
    t_ctr_0, t_ctr_1, t_ctr_2, t_ctr_3, t_ctr_4,
    s_cls_0, s_cls_1, s_cls_2, s_cls_3, s_cls_4,
    s_bbox_0, s_bbox_1, s_bbox_2, s_bbox_3, s_bbox_4,
    s_angle_0, s_angle_1, s_angle_2, s_angle_3, s_angle_4,
    s_ctr_0, s_ctr_1, s_ctr_2, s_ctr_3, s_ctr_4,
):
    tc1, tcsm, tbb1, tbbsm, tct1, tctsm = _pack_side(
        [t_cls_0, t_cls_1, t_cls_2, t_cls_3, t_cls_4],
        [t_bbox_0, t_bbox_1, t_bbox_2, t_bbox_3, t_bbox_4],
        [t_angle_0, t_angle_1, t_angle_2, t_angle_3, t_angle_4],
        [t_ctr_0, t_ctr_1, t_ctr_2, t_ctr_3, t_ctr_4])
    sc1, scsm, sbb1, sbbsm, sct1, sctsm = _pack_side(
        [s_cls_0, s_cls_1, s_cls_2, s_cls_3, s_cls_4],
        [s_bbox_0, s_bbox_1, s_bbox_2, s_bbox_3, s_bbox_4],
        [s_angle_0, s_angle_1, s_angle_2, s_angle_3, s_angle_4],
        [s_ctr_0, s_ctr_1, s_ctr_2, s_ctr_3, s_ctr_4])

    v0, v1, vsm = pl.pallas_call(
        _vbody,
        grid=(_B,),
        in_specs=[
            pl.BlockSpec((1, _C, _H0, _W0), lambda b: (b, 0, 0, 0)),
            pl.BlockSpec((1, _C, _N1), lambda b: (b, 0, 0)),
            pl.BlockSpec((1, _C, _NSM), lambda b: (b, 0, 0)),
        ],
        out_specs=[
            pl.BlockSpec((1, _H0, _W0), lambda b: (b, 0, 0)),
            pl.BlockSpec((1, 1, _N1), lambda b: (b, 0, 0)),
            pl.BlockSpec((1, 1, _NSM), lambda b: (b, 0, 0)),
        ],
        out_shape=[
            jax.ShapeDtypeStruct((_B, _H0, _W0), jnp.float32),
            jax.ShapeDtypeStruct((_B, 1, _N1), jnp.float32),
            jax.ShapeDtypeStruct((_B, 1, _NSM), jnp.float32),
        ],
    )(t_cls_0, tc1, tcsm)

    def bm4(i):
        return (jnp.clip(i - 1, 0, _B - 1), 0, 0, 0)

    def bm3(i):
        return (jnp.clip(i - 1, 0, _B - 1), 0, 0)

    loss_cls, loss_bbox, loss_ctr = pl.pallas_call(
        _lbody,
        grid=(_B + 1,),
        in_specs=[
            pl.BlockSpec((_B, _H0, _W0), lambda i: (0, 0, 0)),
            pl.BlockSpec((_B, 1, _N1), lambda i: (0, 0, 0)),
            pl.BlockSpec((_B, 1, _NSM), lambda i: (0, 0, 0)),
            pl.BlockSpec((1, _C, _H0, _W0), bm4),
            pl.BlockSpec((1, _C, _H0, _W0), bm4),
            pl.BlockSpec((1, 4, _H0, _W0), bm4),
            pl.BlockSpec((1, 1, _H0, _W0), bm4),
            pl.BlockSpec((1, 1, _H0, _W0), bm4),
            pl.BlockSpec((1, 4, _H0, _W0), bm4),
            pl.BlockSpec((1, 1, _H0, _W0), bm4),
            pl.BlockSpec((1, 1, _H0, _W0), bm4),
            pl.BlockSpec((1, _C, _N1), bm3),
            pl.BlockSpec((1, _C, _N1), bm3),
            pl.BlockSpec((1, 5, _N1), bm3),
            pl.BlockSpec((1, 1, _N1), bm3),
            pl.BlockSpec((1, 5, _N1), bm3),
            pl.BlockSpec((1, 1, _N1), bm3),
            pl.BlockSpec((1, _C, _NSM), bm3),
            pl.BlockSpec((1, _C, _NSM), bm3),
            pl.BlockSpec((1, 5, _NSM), bm3),
            pl.BlockSpec((1, 1, _NSM), bm3),
            pl.BlockSpec((1, 5, _NSM), bm3),
            pl.BlockSpec((1, 1, _NSM), bm3),
        ],
        out_specs=[pl.BlockSpec(memory_space=pltpu.SMEM)] * 3,
        out_shape=[jax.ShapeDtypeStruct((1, 1), jnp.float32)] * 3,
        scratch_shapes=[pltpu.SMEM((4,), jnp.int32),
                        pltpu.SMEM((4,), jnp.float32)],
    )(v0, v1, vsm,
      t_cls_0, s_cls_0, t_bbox_0, t_angle_0, t_ctr_0,
      s_bbox_0, s_angle_0, s_ctr_0,
      tc1, sc1, tbb1, tct1, sbb1, sct1,
      tcsm, scsm, tbbsm, tctsm, sbbsm, sctsm)
    return (loss_cls.reshape(()), loss_bbox.reshape(()), loss_ctr.reshape(()))


# speedup vs baseline: 1.2961x; 1.2961x over previous
import jax
import jax.numpy as jnp
from jax import lax
from jax.experimental import pallas as pl
from jax.experimental.pallas import tpu as pltpu

_B = 16
_C = 16
_SZ = ((64, 64), (32, 32), (16, 16), (8, 8), (4, 4))
_NPB = 5456
_N = _B * _NPB
_K = int(_N * 0.01)
_OFF = (0, 4096, 5120, 5376, 5440)


def _smooth_l1(x, y):
    d = jnp.abs(x - y)
    return jnp.where(d < 1.0, 0.5 * d * d, d - 0.5)


def _softplus_p(x):
    e = jnp.exp(-jnp.abs(x))
    a = 1.0 + e
    p = jnp.where(x >= 0.0, 1.0, e) / a
    sp = jnp.maximum(x, 0.0) + jnp.log(a)
    return sp, p


def _body(*refs):
    t_cls = refs[0:5]
    s_cls = refs[5:10]
    t_bbox = refs[10:15]
    t_angle = refs[15:20]
    t_ctr = refs[20:25]
    s_bbox = refs[25:30]
    s_angle = refs[30:35]
    s_ctr = refs[35:40]
    out_cls, out_bbox, out_ctr = refs[40:43]
    v = refs[43:48]
    pnr = refs[48:53]
    pbb = refs[53:58]
    pct = refs[58:63]
    sdf = refs[63]

    i = pl.program_id(0)

    @pl.when(i == 0)
    def _init():
        sdf[1] = jnp.float32(0.0)

    @pl.when(i < _B)
    def _stream():
        b = i
        neg_acc = jnp.float32(0.0)
        for l in range(5):
            tc = t_cls[l][...]
            x = s_cls[l][...]
            sp, p = _softplus_p(x)
            t = jax.nn.sigmoid(tc)
            d = t - p
            pos = (sp - t * x) * (d * d)
            neg = sp * (p * p)
            neg_acc = neg_acc + jnp.sum(neg)
            v[l][pl.ds(b, 1)] = jax.nn.sigmoid(jnp.max(tc, axis=1))
            pnr[l][pl.ds(b, 1)] = jnp.sum(pos - neg, axis=1)

            sl1 = jnp.sum(_smooth_l1(s_bbox[l][...], t_bbox[l][...]), axis=1)
            sl1 = sl1 + _smooth_l1(s_angle[l][...][:, 0],
                                   t_angle[l][...][:, 0])
            w = jax.nn.sigmoid(t_ctr[l][...][:, 0])
            pbb[l][pl.ds(b, 1)] = sl1 * w

            xs = s_ctr[l][...][:, 0]
            es = jnp.exp(-jnp.abs(xs))
            sps = jnp.maximum(xs, 0.0) + jnp.log(1.0 + es)
            pct[l][pl.ds(b, 1)] = sps - w * xs
        sdf[1] += neg_acc

    @pl.when(i == _B)
    def _finish():
        keys = [lax.bitcast_convert_type(v[l][...], jnp.int32)
                for l in range(5)]

        def cnt_gt(x):
            c = jnp.int32(0)
            for k in keys:
                c = c + jnp.sum((k > x).astype(jnp.int32))
            return c

        def bstep(_, lohi):
            lo, hi = lohi
            mid = lo + (hi - lo) // 2
            take_hi = cnt_gt(mid) < _K
            return (jnp.where(take_hi, lo, mid), jnp.where(take_hi, mid, hi))

        _, t_key = lax.fori_loop(0, 31, bstep,
                                 (jnp.int32(-1), jnp.int32(0x3F800000)))
        r = _K - cnt_gt(t_key)

        idxs = []
        for l in range(5):
            sh = keys[l].shape
            idxs.append(lax.broadcasted_iota(jnp.int32, sh, 0) * _NPB
                        + _OFF[l]
                        + lax.broadcasted_iota(jnp.int32, sh, 1) * _SZ[l][1]
                        + lax.broadcasted_iota(jnp.int32, sh, 2))

        def cnt_eq_le(x):
            c = jnp.int32(0)
            for k, ix in zip(keys, idxs):
                c = c + jnp.sum(((k == t_key) & (ix <= x)).astype(jnp.int32))
            return c

        def istep(_, lohi):
            lo, hi = lohi
            mid = lo + (hi - lo) // 2
            enough = cnt_eq_le(mid) >= r
            return (jnp.where(enough, lo, mid), jnp.where(enough, mid, hi))

        _, i_star = lax.fori_loop(0, 17, istep,
                                  (jnp.int32(-1), jnp.int32(_N - 1)))

        t_val = lax.bitcast_convert_type(t_key, jnp.float32)
        fg = t_val * r.astype(jnp.float32)
        acc_cls = jnp.float32(0.0)
        acc_bbox = jnp.float32(0.0)
        acc_ctr = jnp.float32(0.0)
        for l in range(5):
            gt = keys[l] > t_key
            mask = gt | ((keys[l] == t_key) & (idxs[l] <= i_star))
            fg = fg + jnp.sum(jnp.where(gt, v[l][...], 0.0))
            acc_cls = acc_cls + jnp.sum(jnp.where(mask, pnr[l][...], 0.0))
            acc_bbox = acc_bbox + jnp.sum(jnp.where(mask, pbb[l][...], 0.0))
            acc_ctr = acc_ctr + jnp.sum(jnp.where(mask, pct[l][...], 0.0))

        out_cls[0, 0] = (sdf[1] + acc_cls) / fg
        out_bbox[0, 0] = acc_bbox / jnp.float32(_K * 5)
        out_ctr[0, 0] = acc_ctr / jnp.float32(_K)


@jax.jit
def kernel(
    t_cls_0, t_cls_1, t_cls_2, t_cls_3, t_cls_4,
    t_bbox_0, t_bbox_1, t_bbox_2, t_bbox_3, t_bbox_4,
    t_angle_0, t_angle_1, t_angle_2, t_angle_3, t_angle_4,
    t_ctr_0, t_ctr_1, t_ctr_2, t_ctr_3, t_ctr_4,
    s_cls_0, s_cls_1, s_cls_2, s_cls_3, s_cls_4,
    s_bbox_0, s_bbox_1, s_bbox_2, s_bbox_3, s_bbox_4,
    s_angle_0, s_angle_1, s_angle_2, s_angle_3, s_angle_4,
    s_ctr_0, s_ctr_1, s_ctr_2, s_ctr_3, s_ctr_4,
):
    def bm4(i):
        return (jnp.clip(i, 0, _B - 1), 0, 0, 0)

    blk_cls = [pl.BlockSpec((1, _C, h, w), bm4) for h, w in _SZ]
    blk_bb = [pl.BlockSpec((1, 4, h, w), bm4) for h, w in _SZ]
    blk_1 = [pl.BlockSpec((1, 1, h, w), bm4) for h, w in _SZ]

    scr3 = [pltpu.VMEM((_B, h, w), jnp.float32) for h, w in _SZ]

    loss_cls, loss_bbox, loss_ctr = pl.pallas_call(
        _body,
        grid=(_B + 1,),
        in_specs=(blk_cls + blk_cls + blk_bb + blk_1 + blk_1
                  + blk_bb + blk_1 + blk_1),
        out_specs=[pl.BlockSpec(memory_space=pltpu.SMEM)] * 3,
        out_shape=[jax.ShapeDtypeStruct((1, 1), jnp.float32)] * 3,
        scratch_shapes=(scr3 + scr3 + scr3 + scr3
                        + [pltpu.SMEM((4,), jnp.float32)]),
    )(t_cls_0, t_cls_1, t_cls_2, t_cls_3, t_cls_4,
      s_cls_0, s_cls_1, s_cls_2, s_cls_3, s_cls_4,
      t_bbox_0, t_bbox_1, t_bbox_2, t_bbox_3, t_bbox_4,
      t_angle_0, t_angle_1, t_angle_2, t_angle_3, t_angle_4,
      t_ctr_0, t_ctr_1, t_ctr_2, t_ctr_3, t_ctr_4,
      s_bbox_0, s_bbox_1, s_bbox_2, s_bbox_3, s_bbox_4,
      s_angle_0, s_angle_1, s_angle_2, s_angle_3, s_angle_4,
      s_ctr_0, s_ctr_1, s_ctr_2, s_ctr_3, s_ctr_4)
    return (loss_cls.reshape(()), loss_bbox.reshape(()), loss_ctr.reshape(()))


# speedup vs baseline: 1.4800x; 1.1419x over previous
import jax
import jax.numpy as jnp
from jax import lax
from jax.experimental import pallas as pl
from jax.experimental.pallas import tpu as pltpu

_B = 16
_C = 16
_SZ = ((64, 64), (32, 32), (16, 16), (8, 8), (4, 4))
_NPB = 5456
_N = _B * _NPB
_K = int(_N * 0.01)
_OFF = (0, 4096, 5120, 5376, 5440)


def _smooth_l1(x, y):
    d = jnp.abs(x - y)
    return jnp.where(d < 1.0, 0.5 * d * d, d - 0.5)


def _softplus_p(x):
    e = jnp.exp(-jnp.abs(x))
    a = 1.0 + e
    p = jnp.where(x >= 0.0, 1.0, e) / a
    sp = jnp.maximum(x, 0.0) + jnp.log(a)
    return sp, p


def _pack_rows(val, sent):
    del sent
    h = val.shape[1]
    w = val.shape[2]
    group = 128 // w
    pieces = [val[:, j * (h // group):(j + 1) * (h // group), :]
              for j in range(group)]
    return jnp.concatenate(pieces, axis=2)


def _packed_idx(shape, l):
    yi = lax.broadcasted_iota(jnp.int32, shape, 1)
    xi = lax.broadcasted_iota(jnp.int32, shape, 2)
    if l == 0:
        return (yi + 32 * (xi // 64)) * 64 + (xi % 64)
    if l == 1:
        return _OFF[1] + (yi + 8 * (xi // 32)) * 32 + (xi % 32)
    if l == 2:
        return _OFF[2] + (yi + 2 * (xi // 16)) * 16 + (xi % 16)
    return _OFF[3] + xi


def _body(*refs):
    t_cls = refs[0:5]
    s_cls = refs[5:10]
    t_bbox = refs[10:15]
    t_angle = refs[15:20]
    t_ctr = refs[20:25]
    s_bbox = refs[25:30]
    s_angle = refs[30:35]
    s_ctr = refs[35:40]
    out_cls, out_bbox, out_ctr = refs[40:43]
    v = refs[43:47]
    pnr = refs[47:51]
    pbb = refs[51:55]
    pct = refs[55:59]
    sdf = refs[59]

    i = pl.program_id(0)

    @pl.when(i == 0)
    def _init():
        sdf[1] = jnp.float32(0.0)

    @pl.when(i < _B)
    def _stream():
        b = i
        neg_acc = jnp.float32(0.0)
        rows_v = []
        rows_pnr = []
        rows_bb = []
        rows_ct = []
        for l in range(5):
            tc = t_cls[l][...]
            x = s_cls[l][...]
            sp, p = _softplus_p(x)
            t = jax.nn.sigmoid(tc)
            d = t - p
            pos = (sp - t * x) * (d * d)
            neg = sp * (p * p)
            neg_acc = neg_acc + jnp.sum(neg)
            rows_v.append(jax.nn.sigmoid(jnp.max(tc, axis=1)))
            rows_pnr.append(jnp.sum(pos - neg, axis=1))

            sl1 = jnp.sum(_smooth_l1(s_bbox[l][...], t_bbox[l][...]), axis=1)
            sl1 = sl1 + _smooth_l1(s_angle[l][...][:, 0],
                                   t_angle[l][...][:, 0])
            w = jax.nn.sigmoid(t_ctr[l][...][:, 0])
            rows_bb.append(sl1 * w)

            xs = s_ctr[l][...][:, 0]
            es = jnp.exp(-jnp.abs(xs))
            sps = jnp.maximum(xs, 0.0) + jnp.log(1.0 + es)
            rows_ct.append(sps - w * xs)
        sdf[1] += neg_acc

        def flat(val):
            h, w = val.shape[1], val.shape[2]
            return jnp.concatenate(
                [val[:, j:j + 1, :] for j in range(h)], axis=2)

        for dst, rows, sent in ((v, rows_v, -1.0), (pnr, rows_pnr, 0.0),
                                (pbb, rows_bb, 0.0), (pct, rows_ct, 0.0)):
            for l in range(3):
                dst[l][pl.ds(b, 1)] = _pack_rows(rows[l], sent)
            tail = jnp.concatenate(
                [flat(rows[3]), flat(rows[4]),
                 jnp.full((1, 1, 48), sent, jnp.float32)], axis=2)
            dst[3][pl.ds(b, 1)] = tail

    @pl.when(i == _B)
    def _finish():
        keys = [lax.bitcast_convert_type(v[g][...], jnp.int32)
                for g in range(4)]

        def cnt_gt(x):
            c = jnp.int32(0)
            for k in keys:
                c = c + jnp.sum((k > x).astype(jnp.int32))
            return c

        def bstep(_, lohi):
            lo, hi = lohi
            mid = lo + (hi - lo) // 2
            take_hi = cnt_gt(mid) < _K
            return (jnp.where(take_hi, lo, mid), jnp.where(take_hi, mid, hi))

        _, t_key = lax.fori_loop(0, 31, bstep,
                                 (jnp.int32(-1), jnp.int32(0x3F800000)))
        r = _K - cnt_gt(t_key)

        idxs = [lax.broadcasted_iota(jnp.int32, keys[g].shape, 0) * _NPB
                + _packed_idx(keys[g].shape, g) for g in range(4)]

        def cnt_eq_le(x):
            c = jnp.int32(0)
            for k, ix in zip(keys, idxs):
                c = c + jnp.sum(((k == t_key) & (ix <= x)).astype(jnp.int32))
            return c

        def istep(_, lohi):
            lo, hi = lohi
            mid = lo + (hi - lo) // 2
            enough = cnt_eq_le(mid) >= r
            return (jnp.where(enough, lo, mid), jnp.where(enough, mid, hi))

        _, i_star = lax.fori_loop(0, 17, istep,
                                  (jnp.int32(-1), jnp.int32(_N - 1)))

        t_val = lax.bitcast_convert_type(t_key, jnp.float32)
        fg = t_val * r.astype(jnp.float32)
        acc_cls = jnp.float32(0.0)
        acc_bbox = jnp.float32(0.0)
        acc_ctr = jnp.float32(0.0)
        for g in range(4):
            gt = keys[g] > t_key
            mask = gt | ((keys[g] == t_key) & (idxs[g] <= i_star))
            fg = fg + jnp.sum(jnp.where(gt, v[g][...], 0.0))
            acc_cls = acc_cls + jnp.sum(jnp.where(mask, pnr[g][...], 0.0))
            acc_bbox = acc_bbox + jnp.sum(jnp.where(mask, pbb[g][...], 0.0))
            acc_ctr = acc_ctr + jnp.sum(jnp.where(mask, pct[g][...], 0.0))

        out_cls[0, 0] = (sdf[1] + acc_cls) / fg
        out_bbox[0, 0] = acc_bbox / jnp.float32(_K * 5)
        out_ctr[0, 0] = acc_ctr / jnp.float32(_K)


@jax.jit
def kernel(
    t_cls_0, t_cls_1, t_cls_2, t_cls_3, t_cls_4,
    t_bbox_0, t_bbox_1, t_bbox_2, t_bbox_3, t_bbox_4,
    t_angle_0, t_angle_1, t_angle_2, t_angle_3, t_angle_4,
    t_ctr_0, t_ctr_1, t_ctr_2, t_ctr_3, t_ctr_4,
    s_cls_0, s_cls_1, s_cls_2, s_cls_3, s_cls_4,
    s_bbox_0, s_bbox_1, s_bbox_2, s_bbox_3, s_bbox_4,
    s_angle_0, s_angle_1, s_angle_2, s_angle_3, s_angle_4,
    s_ctr_0, s_ctr_1, s_ctr_2, s_ctr_3, s_ctr_4,
):
    def bm4(i):
        return (jnp.clip(i, 0, _B - 1), 0, 0, 0)

    blk_cls = [pl.BlockSpec((1, _C, h, w), bm4) for h, w in _SZ]
    blk_bb = [pl.BlockSpec((1, 4, h, w), bm4) for h, w in _SZ]
    blk_1 = [pl.BlockSpec((1, 1, h, w), bm4) for h, w in _SZ]

    scr3 = [pltpu.VMEM((_B, 32, 128), jnp.float32),
            pltpu.VMEM((_B, 8, 128), jnp.float32),
            pltpu.VMEM((_B, 2, 128), jnp.float32),
            pltpu.VMEM((_B, 1, 128), jnp.float32)]

    loss_cls, loss_bbox, loss_ctr = pl.pallas_call(
        _body,
        grid=(_B + 1,),
        in_specs=(blk_cls + blk_cls + blk_bb + blk_1 + blk_1
                  + blk_bb + blk_1 + blk_1),
        out_specs=[pl.BlockSpec(memory_space=pltpu.SMEM)] * 3,
        out_shape=[jax.ShapeDtypeStruct((1, 1), jnp.float32)] * 3,
        scratch_shapes=(scr3 + scr3 + scr3 + scr3
                        + [pltpu.SMEM((4,), jnp.float32)]),
    )(t_cls_0, t_cls_1, t_cls_2, t_cls_3, t_cls_4,
      s_cls_0, s_cls_1, s_cls_2, s_cls_3, s_cls_4,
      t_bbox_0, t_bbox_1, t_bbox_2, t_bbox_3, t_bbox_4,
      t_angle_0, t_angle_1, t_angle_2, t_angle_3, t_angle_4,
      t_ctr_0, t_ctr_1, t_ctr_2, t_ctr_3, t_ctr_4,
      s_bbox_0, s_bbox_1, s_bbox_2, s_bbox_3, s_bbox_4,
      s_angle_0, s_angle_1, s_angle_2, s_angle_3, s_angle_4,
      s_ctr_0, s_ctr_1, s_ctr_2, s_ctr_3, s_ctr_4)
    return (loss_cls.reshape(()), loss_bbox.reshape(()), loss_ctr.reshape(()))
